# baseline (speedup 1.0000x reference)
import functools
import math

import jax
import jax.numpy as jnp
from jax import lax
from jax.experimental import pallas as pl
from jax.experimental.pallas import tpu as pltpu
from jax.experimental.pallas import tpu_sc as plsc

D_LANES = 16


def _make_gather(n_total: int, d: int, vocab: int, chunk: int):
  info = plsc.get_sparse_core_info()
  nc, ns = info.num_cores, info.num_subcores
  nw = nc * ns
  assert n_total % nw == 0
  per_w = n_total // nw
  assert per_w % chunk == 0
  n_chunks = per_w // chunk
  mesh = plsc.VectorSubcoreMesh(core_axis_name="c", subcore_axis_name="s")

  @functools.partial(
      pl.kernel,
      mesh=mesh,
      compiler_params=pltpu.CompilerParams(use_tc_tiling_on_sc=False),
      out_type=jax.ShapeDtypeStruct((n_total, d), jnp.float32),
      scratch_types=[
          pltpu.VMEM((chunk,), jnp.int32),
          pltpu.VMEM((chunk, d), jnp.float32),
          pltpu.SemaphoreType.DMA,
      ],
  )
  def gather_k(lut_hbm, idx_hbm, out_hbm, idx_v, rows_v, sem):
    wid = lax.axis_index("s") * nc + lax.axis_index("c")
    base = wid * per_w

    def chunk_body(g, carry):
      off = base + g * chunk
      pltpu.sync_copy(idx_hbm.at[pl.ds(off, chunk)], idx_v)
      pltpu.async_copy(lut_hbm.at[idx_v], rows_v, sem).wait()

      def scale_row(r, c2):
        for j in range(d // D_LANES):
          sl = pl.ds(j * D_LANES, D_LANES)
          rows_v[r, sl] = rows_v[r, sl] * 8.0
        return c2

      lax.fori_loop(0, chunk, scale_row, 0, unroll=2)
      pltpu.sync_copy(rows_v, out_hbm.at[pl.ds(off, chunk)])
      return carry

    lax.fori_loop(0, n_chunks, chunk_body, 0)

  return gather_k


def kernel(x, lut):
  b, s = x.shape
  vocab, d = lut.shape
  n_total = b * s
  gather_k = _make_gather(n_total, d, vocab, chunk=512)
  out = gather_k(lut, x.reshape(n_total))
  return out.reshape(b, s, d)


# speedup vs baseline: 1.0913x; 1.0913x over previous
import functools

import jax
import jax.numpy as jnp
from jax import lax
from jax.experimental import pallas as pl
from jax.experimental.pallas import tpu as pltpu
from jax.experimental.pallas import tpu_sc as plsc

D_LANES = 16


def _make_gather(n_total: int, d: int, chunk: int):
  info = plsc.get_sparse_core_info()
  nc, ns = info.num_cores, info.num_subcores
  nw = nc * ns
  assert n_total % nw == 0
  per_w = n_total // nw
  assert per_w % (2 * chunk) == 0
  n_groups = per_w // (2 * chunk)
  mesh = plsc.VectorSubcoreMesh(core_axis_name="c", subcore_axis_name="s")

  @functools.partial(
      pl.kernel,
      mesh=mesh,
      compiler_params=pltpu.CompilerParams(use_tc_tiling_on_sc=False),
      out_type=jax.ShapeDtypeStruct((n_total, d), jnp.float32),
      scratch_types=[
          pltpu.VMEM((per_w,), jnp.int32),
          pltpu.VMEM((chunk, d), jnp.float32),
          pltpu.VMEM((chunk, d), jnp.float32),
          pltpu.SemaphoreType.DMA,
          pltpu.SemaphoreType.DMA,
          pltpu.SemaphoreType.DMA,
          pltpu.SemaphoreType.DMA,
      ],
  )
  def gather_k(lut_hbm, idx_hbm, out_hbm, idx_all, rows0, rows1,
               gsem0, gsem1, wsem0, wsem1):
    wid = lax.axis_index("s") * nc + lax.axis_index("c")
    base = wid * per_w
    pltpu.sync_copy(idx_hbm.at[pl.ds(base, per_w)], idx_all)

    def gather_cp(g, rows, gsem):
      return pltpu.make_async_copy(
          lut_hbm.at[idx_all.at[pl.ds(g * chunk, chunk)]], rows, gsem)

    def wb_cp(g, rows, wsem):
      return pltpu.make_async_copy(
          rows, out_hbm.at[pl.ds(base + g * chunk, chunk)], wsem)

    def scale(rows):
      def scale_row(r, c2):
        for j in range(d // D_LANES):
          sl = pl.ds(j * D_LANES, D_LANES)
          rows[r, sl] = rows[r, sl] * 8.0
        return c2
      lax.fori_loop(0, chunk, scale_row, 0, unroll=2)

    gather_cp(0, rows0, gsem0).start()

    def group(t, carry):
      g0 = 2 * t
      g1 = g0 + 1
      @pl.when(t > 0)
      def _():
        wb_cp(g0 - 1, rows1, wsem1).wait()
      gather_cp(g1, rows1, gsem1).start()
      gather_cp(g0, rows0, gsem0).wait()
      scale(rows0)
      wb_cp(g0, rows0, wsem0).start()
      gather_cp(g1, rows1, gsem1).wait()
      @pl.when(t < n_groups - 1)
      def _():
        wb_cp(g0, rows0, wsem0).wait()
        gather_cp(g0 + 2, rows0, gsem0).start()
      scale(rows1)
      wb_cp(g1, rows1, wsem1).start()
      return carry

    lax.fori_loop(0, n_groups, group, 0)
    wb_cp(2 * n_groups - 2, rows0, wsem0).wait()
    wb_cp(2 * n_groups - 1, rows1, wsem1).wait()

  return gather_k


def kernel(x, lut):
  b, s = x.shape
  _, d = lut.shape
  n_total = b * s
  gather_k = _make_gather(n_total, d, chunk=512)
  out = gather_k(lut, x.reshape(n_total))
  return out.reshape(b, s, d)
